# baseline (speedup 1.0000x reference)
import jax
import jax.numpy as jnp
from jax import lax
from jax.experimental import pallas as pl
from jax.experimental.pallas import tpu as pltpu
from jax.experimental.pallas import tpu_sc as plsc

_B, _S, _D = 4, 4096, 1024
_NC, _NS = 2, 16
_BLK = 128
_TR = _BLK // _NS
_NK = _S // _BLK
_NR = _NK * 2
_SR = 5
_TB = 3


def _sc_body(x_hbm, pe_hbm, out_hbm, *refs):
    sps = refs[0:_SR]
    pss = refs[_SR:_SR + 2]
    pbs = refs[_SR + 2:_SR + 4]
    tbs = refs[_SR + 4:_SR + 4 + _TB]
    base = _SR + 4 + _TB
    sfs = refs[base:base + _SR]
    spfs = refs[base + _SR:base + _SR + 2]
    spes = refs[base + _SR + 2:base + _SR + 4]
    sxis = refs[base + _SR + 4:base + _SR + 4 + _TB]
    sots = refs[base + _SR + 4 + _TB:base + _SR + 4 + 2 * _TB]

    c = lax.axis_index("c")
    tid = lax.axis_index("s")

    def row0(t):
        k, b = t // 2, t % 2
        return (c * 2 + b) * _S + k * _BLK + tid * _TR

    def myslice(i):
        return sps[i].at[pl.ds(tid * _TR, _TR)]

    def fill(t):
        pltpu.async_copy(x_hbm.at[pl.ds(row0(t), _TR)], myslice(t % _SR),
                         sfs[t % _SR])

    def peslice(i):
        return pss[i].at[pl.ds(tid * _TR, _TR)]

    def fill_pe(k):
        pltpu.async_copy(pe_hbm.at[pl.ds(k * _BLK + tid * _TR, _TR)],
                         peslice(k % 2), spfs[k % 2])

    def pull_pe(k):
        pltpu.async_copy(peslice(k % 2), pbs[k % 2], spes[k % 2])

    def pull(t):
        pltpu.async_copy(myslice(t % _SR), tbs[t % _TB], sxis[t % _TB])

    def wait_in(ref, sem):
        pltpu.make_async_copy(x_hbm.at[pl.ds(0, _TR)], ref, sem).wait()

    def wait_out(i):
        pltpu.make_async_copy(tbs[i], out_hbm.at[pl.ds(0, _TR)],
                              sots[i]).wait()

    fill(0)
    fill(1)
    fill(2)
    fill_pe(0)
    fill_pe(1)
    wait_in(myslice(0), sfs[0])
    pull(0)
    wait_in(peslice(0), spfs[0])
    pull_pe(0)
    for t in range(_NR):
        k, b = t // 2, t % 2
        if t + 3 < _NR:
            fill(t + 3)
        if t + 1 < _NR:
            if t >= 2:
                wait_out((t + 1) % _TB)
            wait_in(myslice((t + 1) % _SR), sfs[(t + 1) % _SR])
            pull(t + 1)
        wait_in(tbs[t % _TB], sxis[t % _TB])
        if b == 0:
            wait_in(pbs[k % 2], spes[k % 2])
            if k + 2 < _NK:
                fill_pe(k + 2)
        if b == 1 and k + 1 < _NK:
            wait_in(peslice((k + 1) % 2), spfs[(k + 1) % 2])
            pull_pe(k + 1)
        tb, pb = tbs[t % _TB], pbs[k % 2]

        def row_body(r, carry, tb=tb, pb=pb):
            @plsc.parallel_loop(0, _D, step=16, unroll=8)
            def _add16(i):
                plsc.addupdate(tb.at[r, pl.ds(i, 16)], pb[r, pl.ds(i, 16)])
            return carry

        lax.fori_loop(0, _TR, row_body, 0)
        pltpu.async_copy(tb, out_hbm.at[pl.ds(row0(t), _TR)], sots[t % _TB])
    for i in range(_TB):
        wait_out(i)


def kernel(x, pos_embedding):
    B, S, D = x.shape
    x2 = x.reshape(B * S, D)
    mesh = plsc.VectorSubcoreMesh(core_axis_name="c", subcore_axis_name="s")
    out = pl.kernel(
        _sc_body,
        out_type=jax.ShapeDtypeStruct((B * S, D), x.dtype),
        mesh=mesh,
        scratch_types=(
            [pltpu.VMEM_SHARED((_BLK, _D), jnp.float32)] * (_SR + 2)
            + [pltpu.VMEM((_TR, _D), jnp.float32)] * (2 + _TB)
            + [pltpu.SemaphoreType.DMA] * (_SR + 4 + 2 * _TB)
        ),
    )(x2, pos_embedding)
    return out.reshape(B, S, D)
